# baseline (speedup 1.0000x reference)
import functools

import jax
import jax.numpy as jnp
from jax import lax
from jax.experimental import pallas as pl
from jax.experimental.pallas import tpu as pltpu
from jax.experimental.pallas import tpu_sc as plsc

VOCAB = 100000
EMBED = 128
HIDDEN = 256
BATCH = 4096
SEQ = 200

NC = 2
NS = 16
NW = NC * NS
BPW = BATCH // NW
LANES = 16
NV = EMBED // LANES
SEQ_A = 128
SEQ_B = SEQ - SEQ_A
NBUF = 3


def _accum_row(rows_v, buf, b, pooled_v):
    def body(s, acc):
        return tuple(acc[j] + rows_v[buf, s, pl.ds(LANES * j, LANES)]
                     for j in range(NV))

    init = tuple(jnp.zeros((LANES,), jnp.float32) for _ in range(NV))
    acc = lax.fori_loop(0, SEQ, body, init, unroll=8)
    for j in range(NV):
        pooled_v[b, pl.ds(LANES * j, LANES)] = acc[j]


def _sc_body(idx_a_hbm, idx_b_hbm, table_hbm, out_hbm,
             idx_a_v, idx_b_v, rows_v, pooled_v, *sems):
    c = lax.axis_index("c")
    s = lax.axis_index("s")
    wid = s * NC + c

    pltpu.sync_copy(idx_a_hbm.at[wid], idx_a_v)
    pltpu.sync_copy(idx_b_hbm.at[wid], idx_b_v)

    def copies(b, buf):
        return (
            pltpu.make_async_copy(
                table_hbm.at[idx_a_v.at[b]],
                rows_v.at[buf].at[pl.ds(0, SEQ_A)],
                sems[buf]),
            pltpu.make_async_copy(
                table_hbm.at[idx_b_v.at[b]],
                rows_v.at[buf].at[pl.ds(SEQ_A, SEQ_B)],
                sems[buf]),
        )

    def issue(b, buf):
        for cpy in copies(b, buf):
            cpy.start()

    def drain(b, buf):
        for cpy in copies(b, buf):
            cpy.wait()

    for k in range(NBUF):
        issue(k, k)

    def outer(i, carry):
        for k in range(NBUF):
            b = NBUF * i + k
            drain(b, k)
            _accum_row(rows_v, k, b, pooled_v)

            @pl.when(b + NBUF < BPW)
            def _():
                issue(b + NBUF, k)

        return carry

    lax.fori_loop(0, BPW // NBUF, outer, 0)
    for b in range(BPW - BPW % NBUF, BPW):
        k = b % NBUF
        drain(b, k)
        _accum_row(rows_v, k, b, pooled_v)

    pltpu.sync_copy(pooled_v, out_hbm.at[pl.ds(wid * BPW, BPW)])


@functools.cache
def _gather_sum():
    return pl.kernel(
        _sc_body,
        out_type=jax.ShapeDtypeStruct((BATCH, EMBED), jnp.float32),
        mesh=plsc.VectorSubcoreMesh(core_axis_name="c", subcore_axis_name="s",
                                    num_cores=NC, num_subcores=NS),
        scratch_types=[
            pltpu.VMEM((BPW, SEQ_A), jnp.int32),
            pltpu.VMEM((BPW, SEQ_B), jnp.int32),
            pltpu.VMEM((NBUF, SEQ, EMBED), jnp.float32),
            pltpu.VMEM((BPW, EMBED), jnp.float32),
        ] + [pltpu.SemaphoreType.DMA] * NBUF,
    )


def _fc_body(p_ref, w_ref, b_ref, o_ref):
    acc = jnp.dot(p_ref[...], w_ref[...], preferred_element_type=jnp.float32)
    o_ref[...] = jnp.maximum(acc + b_ref[...], 0.0)


_TB = 1024

_fc_relu = pl.pallas_call(
    _fc_body,
    grid=(BATCH // _TB,),
    in_specs=[
        pl.BlockSpec((_TB, EMBED), lambda i: (i, 0)),
        pl.BlockSpec((EMBED, HIDDEN), lambda i: (0, 0)),
        pl.BlockSpec((1, HIDDEN), lambda i: (0, 0)),
    ],
    out_specs=pl.BlockSpec((_TB, HIDDEN), lambda i: (i, 0)),
    out_shape=jax.ShapeDtypeStruct((BATCH, HIDDEN), jnp.float32),
)


def kernel(token_ids, emb_table, fc_w, fc_b):
    idx_a = token_ids[:, :SEQ_A].reshape(NW, BPW, SEQ_A)
    idx_b = token_ids[:, SEQ_A:].reshape(NW, BPW, SEQ_B)
    pooled = _gather_sum()(idx_a, idx_b, emb_table)
    wt = fc_w.T * jnp.float32(1.0 / SEQ)
    return _fc_relu(pooled, wt, fc_b.reshape(1, HIDDEN))


# speedup vs baseline: 1.0037x; 1.0037x over previous
import functools

import jax
import jax.numpy as jnp
from jax import lax
from jax.experimental import pallas as pl
from jax.experimental.pallas import tpu as pltpu
from jax.experimental.pallas import tpu_sc as plsc

VOCAB = 100000
EMBED = 128
HIDDEN = 256
BATCH = 4096
SEQ = 200

NC = 2
NS = 16
NW = NC * NS
BPW = BATCH // NW
LANES = 16
NV = EMBED // LANES
SEQ_A = 128
SEQ_B = SEQ - SEQ_A
NBUF = 3


def _accum_row(rows_v, buf, b, pooled_v):
    def body(s, acc):
        return tuple(acc[j] + rows_v[buf, s, pl.ds(LANES * j, LANES)]
                     for j in range(NV))

    init = tuple(jnp.zeros((LANES,), jnp.float32) for _ in range(NV))
    acc = lax.fori_loop(0, SEQ, body, init, unroll=4)
    for j in range(NV):
        pooled_v[b, pl.ds(LANES * j, LANES)] = acc[j]


def _sc_body(idx_hbm, table_hbm, out_hbm,
             idx_v, rows_v, pooled_v, *sems):
    c = lax.axis_index("c")
    s = lax.axis_index("s")
    wid = s * NC + c

    pltpu.sync_copy(idx_hbm.at[pl.ds(wid * BPW, BPW)], idx_v)

    def copies(b, buf):
        return (
            pltpu.make_async_copy(
                table_hbm.at[idx_v.at[b, pl.ds(0, SEQ_A)]],
                rows_v.at[buf].at[pl.ds(0, SEQ_A)],
                sems[buf]),
            pltpu.make_async_copy(
                table_hbm.at[idx_v.at[b, pl.ds(SEQ_A, SEQ_B)]],
                rows_v.at[buf].at[pl.ds(SEQ_A, SEQ_B)],
                sems[buf]),
        )

    def issue(b, buf):
        for cpy in copies(b, buf):
            cpy.start()

    def drain(b, buf):
        for cpy in copies(b, buf):
            cpy.wait()

    for k in range(NBUF):
        issue(k, k)

    def outer(i, carry):
        for k in range(NBUF):
            b = NBUF * i + k
            drain(b, k)
            _accum_row(rows_v, k, b, pooled_v)

            @pl.when(b + NBUF < BPW)
            def _():
                issue(b + NBUF, k)

        return carry

    lax.fori_loop(0, BPW // NBUF, outer, 0)
    for b in range(BPW - BPW % NBUF, BPW):
        k = b % NBUF
        drain(b, k)
        _accum_row(rows_v, k, b, pooled_v)

    pltpu.sync_copy(pooled_v, out_hbm.at[pl.ds(wid * BPW, BPW)])


@functools.cache
def _gather_sum():
    return pl.kernel(
        _sc_body,
        out_type=jax.ShapeDtypeStruct((BATCH, EMBED), jnp.float32),
        mesh=plsc.VectorSubcoreMesh(core_axis_name="c", subcore_axis_name="s",
                                    num_cores=NC, num_subcores=NS),
        scratch_types=[
            pltpu.VMEM((BPW, SEQ), jnp.int32),
            pltpu.VMEM((NBUF, SEQ, EMBED), jnp.float32),
            pltpu.VMEM((BPW, EMBED), jnp.float32),
        ] + [pltpu.SemaphoreType.DMA] * NBUF,
    )


def _fc_body(p_ref, w_ref, b_ref, o_ref):
    acc = jnp.dot(p_ref[...], w_ref[...], preferred_element_type=jnp.float32)
    o_ref[...] = jnp.maximum(acc + b_ref[...], 0.0)


_TB = 1024

_fc_relu = pl.pallas_call(
    _fc_body,
    grid=(BATCH // _TB,),
    in_specs=[
        pl.BlockSpec((_TB, EMBED), lambda i: (i, 0)),
        pl.BlockSpec((EMBED, HIDDEN), lambda i: (0, 0)),
        pl.BlockSpec((1, HIDDEN), lambda i: (0, 0)),
    ],
    out_specs=pl.BlockSpec((_TB, HIDDEN), lambda i: (i, 0)),
    out_shape=jax.ShapeDtypeStruct((BATCH, HIDDEN), jnp.float32),
)


def kernel(token_ids, emb_table, fc_w, fc_b):
    pooled = _gather_sum()(token_ids, emb_table)
    wt = fc_w.T * jnp.float32(1.0 / SEQ)
    return _fc_relu(pooled, wt, fc_b.reshape(1, HIDDEN))


# speedup vs baseline: 1.0045x; 1.0008x over previous
import functools

import jax
import jax.numpy as jnp
from jax import lax
from jax.experimental import pallas as pl
from jax.experimental.pallas import tpu as pltpu
from jax.experimental.pallas import tpu_sc as plsc

VOCAB = 100000
EMBED = 128
HIDDEN = 256
BATCH = 4096
SEQ = 200

NC = 2
NS = 16
NW = NC * NS
BPW = BATCH // NW
LANES = 16
NV = EMBED // LANES
SEQ_A = 128
SEQ_B = SEQ - SEQ_A
NBUF = 3


def _accum_row(rows_v, buf, b, pooled_v):
    def body(s, acc):
        return tuple(acc[j] + rows_v[buf, s, pl.ds(LANES * j, LANES)]
                     for j in range(NV))

    init = tuple(jnp.zeros((LANES,), jnp.float32) for _ in range(NV))
    acc = lax.fori_loop(0, SEQ, body, init, unroll=4)
    for j in range(NV):
        pooled_v[b, pl.ds(LANES * j, LANES)] = acc[j]


def _sc_body(idx_hbm, table_hbm, out_hbm,
             idx_v, rows_v, pooled_v, *sems):
    c = lax.axis_index("c")
    s = lax.axis_index("s")
    wid = s * NC + c

    pltpu.sync_copy(idx_hbm.at[pl.ds(wid * BPW, BPW)], idx_v)

    def copies(b, buf):
        return (
            pltpu.make_async_copy(
                table_hbm.at[idx_v.at[b, pl.ds(0, SEQ_A)]],
                rows_v.at[buf].at[pl.ds(0, SEQ_A)],
                sems[buf]),
            pltpu.make_async_copy(
                table_hbm.at[idx_v.at[b, pl.ds(SEQ_A, SEQ_B)]],
                rows_v.at[buf].at[pl.ds(SEQ_A, SEQ_B)],
                sems[buf]),
        )

    def issue(b, buf):
        for cpy in copies(b, buf):
            cpy.start()

    def drain(b, buf):
        pltpu.make_async_copy(
            table_hbm.at[pl.ds(0, SEQ)], rows_v.at[buf], sems[buf]).wait()

    for k in range(NBUF):
        issue(k, k)

    def outer(i, carry):
        for k in range(NBUF):
            b = NBUF * i + k
            drain(b, k)
            _accum_row(rows_v, k, b, pooled_v)

            @pl.when(b + NBUF < BPW)
            def _():
                issue(b + NBUF, k)

        return carry

    lax.fori_loop(0, BPW // NBUF, outer, 0)
    for b in range(BPW - BPW % NBUF, BPW):
        k = b % NBUF
        drain(b, k)
        _accum_row(rows_v, k, b, pooled_v)

    pltpu.sync_copy(pooled_v, out_hbm.at[pl.ds(wid * BPW, BPW)])


@functools.cache
def _gather_sum():
    return pl.kernel(
        _sc_body,
        out_type=jax.ShapeDtypeStruct((BATCH, EMBED), jnp.float32),
        mesh=plsc.VectorSubcoreMesh(core_axis_name="c", subcore_axis_name="s",
                                    num_cores=NC, num_subcores=NS),
        scratch_types=[
            pltpu.VMEM((BPW, SEQ), jnp.int32),
            pltpu.VMEM((NBUF, SEQ, EMBED), jnp.float32),
            pltpu.VMEM((BPW, EMBED), jnp.float32),
        ] + [pltpu.SemaphoreType.DMA] * NBUF,
    )


def _fc_body(p_ref, w_ref, b_ref, o_ref):
    acc = jnp.dot(p_ref[...], w_ref[...], preferred_element_type=jnp.float32)
    o_ref[...] = jnp.maximum(acc + b_ref[...], 0.0)


_TB = 1024

_fc_relu = pl.pallas_call(
    _fc_body,
    grid=(BATCH // _TB,),
    in_specs=[
        pl.BlockSpec((_TB, EMBED), lambda i: (i, 0)),
        pl.BlockSpec((EMBED, HIDDEN), lambda i: (0, 0)),
        pl.BlockSpec((1, HIDDEN), lambda i: (0, 0)),
    ],
    out_specs=pl.BlockSpec((_TB, HIDDEN), lambda i: (i, 0)),
    out_shape=jax.ShapeDtypeStruct((BATCH, HIDDEN), jnp.float32),
)


def kernel(token_ids, emb_table, fc_w, fc_b):
    pooled = _gather_sum()(token_ids, emb_table)
    wt = fc_w.T * jnp.float32(1.0 / SEQ)
    return _fc_relu(pooled, wt, fc_b.reshape(1, HIDDEN))
